# Initial kernel scaffold; baseline (speedup 1.0000x reference)
import jax
import jax.numpy as jnp
from jax.experimental import pallas as pl


def kernel(parent_feature, W_parent, b_parent, W_exists, b_exists, W_el, b_el, W_ee, b_ee, W_ne, b_ne, W_child, b_child, W_sem, b_sem, W_child2, b_child2):
    raise NotImplementedError("write your pallas kernel here")


# speedup vs baseline: 22.0573x; 22.0573x over previous
import jax
import jax.numpy as jnp
from jax.experimental import pallas as pl
from jax.experimental.pallas import tpu as pltpu

C = 128
H = 256
F = 256
ITERS = 2
ET = 4
NSEM = 57
NCHUNK = 8
CB = C // NCHUNK
COLS = CB * H
IB = 16


def _body(parent_ref, wp_ref, bp_ref, wex_ref, bex_ref, wel_ref, bel_ref,
          wee_ref, bee_ref, wne_ref, bne_ref, wch_ref, bch_ref,
          wsem_ref, bsem_ref, wch2_ref, bch2_ref,
          out_ref, sem_ref, exists_ref, elog_ref,
          child_ref):
    k = pl.program_id(0)

    pf = jnp.dot(parent_ref[...], wp_ref[...],
                 preferred_element_type=jnp.float32) + bp_ref[...][None, :]
    child_ref[pl.ds(k * CB, CB), :] = jnp.maximum(pf, 0.0).reshape(CB, H)

    @pl.when(k == NCHUNK - 1)
    def _rest():
        child = child_ref[...]

        exl = jnp.dot(child, wex_ref[...],
                      preferred_element_type=jnp.float32) + bex_ref[...]
        exists_ref[...] = exl[None]
        exf = (exl[:, 0] > 0.0).astype(jnp.float32)

        wel = wel_ref[...]
        ea = jnp.dot(child, wel[:H],
                     preferred_element_type=jnp.float32) + bel_ref[...]
        eb = jnp.dot(child, wel[H:], preferred_element_type=jnp.float32)
        cnt_rows = []
        for ib in range(C // IB):
            el = jnp.maximum(ea[ib * IB:(ib + 1) * IB][:, None, :]
                             + eb[None, :, :], 0.0)
            lb = jnp.dot(el.reshape(IB * C, H), wee_ref[...],
                         preferred_element_type=jnp.float32) + bee_ref[...]
            lb3 = lb.reshape(IB, C, ET)
            elog_ref[0, pl.ds(ib * IB, IB), :, :] = lb3
            pos = (lb3 > 0.0).astype(jnp.float32).sum(axis=2)
            cnt_rows.append(pos * exf[ib * IB:(ib + 1) * IB][:, None]
                            * exf[None, :])
        cnt = jnp.concatenate(cnt_rows, axis=0)
        has_edges = jnp.any(cnt > 0.0)

        cf = child
        feats = [child]
        for it in range(ITERS):
            a = jnp.dot(cf, wne_ref[it, :H],
                        preferred_element_type=jnp.float32) + bne_ref[it][None, :]
            b = jnp.dot(cf, wne_ref[it, H:], preferred_element_type=jnp.float32)
            rows = []
            for ib in range(C // IB):
                m = jnp.maximum(a[ib * IB:(ib + 1) * IB][:, None, :]
                                + b[None, :, :], 0.0)
                w = cnt[ib * IB:(ib + 1) * IB][:, :, None]
                rows.append(jnp.sum(m * w, axis=1))
            agg = jnp.concatenate(rows, axis=0)
            cf = jnp.where(has_edges, agg, cf)
            feats.append(cf)

        cf3 = jnp.concatenate(feats, axis=1)
        h = jnp.maximum(jnp.dot(cf3, wch_ref[...],
                                preferred_element_type=jnp.float32)
                        + bch_ref[...], 0.0)
        sem_ref[...] = (jnp.dot(h, wsem_ref[...],
                                preferred_element_type=jnp.float32)
                        + bsem_ref[...])[None]
        out_ref[...] = jnp.maximum(jnp.dot(h, wch2_ref[...],
                                           preferred_element_type=jnp.float32)
                                   + bch2_ref[...], 0.0)[None]


def kernel(parent_feature, W_parent, b_parent, W_exists, b_exists, W_el, b_el,
           W_ee, b_ee, W_ne, b_ne, W_child, b_child, W_sem, b_sem,
           W_child2, b_child2):
    f32 = jnp.float32
    wee2 = W_ee[:, :, 0].T
    bee2 = b_ee[:, 0][None, :]
    full = lambda s: pl.BlockSpec(s, lambda k: (0,) * len(s))
    out, sem, exists_logits, elog = pl.pallas_call(
        _body,
        grid=(NCHUNK,),
        in_specs=[
            full((1, F)),
            pl.BlockSpec((F, COLS), lambda k: (0, k)),
            pl.BlockSpec((COLS,), lambda k: (k,)),
            full((H, 1)), full((1, 1)),
            full((2 * H, H)), full((1, H)),
            full((H, ET)), full((1, ET)),
            full((ITERS, 2 * H, H)), full((ITERS, H)),
            full((H * (ITERS + 1), H)), full((1, H)),
            full((H, NSEM)), full((1, NSEM)),
            full((H, F)), full((1, F)),
        ],
        out_specs=[
            full((1, C, F)), full((1, C, NSEM)),
            full((1, C, 1)), full((1, C, C, ET)),
        ],
        out_shape=[
            jax.ShapeDtypeStruct((1, C, F), f32),
            jax.ShapeDtypeStruct((1, C, NSEM), f32),
            jax.ShapeDtypeStruct((1, C, 1), f32),
            jax.ShapeDtypeStruct((1, C, C, ET), f32),
        ],
        scratch_shapes=[pltpu.VMEM((C, H), f32)],
        compiler_params=pltpu.CompilerParams(
            dimension_semantics=("arbitrary",)),
    )(parent_feature, W_parent, b_parent,
      W_exists, b_exists[None, :],
      W_el, b_el[None, :],
      wee2, bee2,
      W_ne, b_ne,
      W_child, b_child[None, :],
      W_sem, b_sem[None, :],
      W_child2, b_child2[None, :])
    return out, sem, exists_logits, elog


# speedup vs baseline: 23.2604x; 1.0545x over previous
import jax
import jax.numpy as jnp
from jax.experimental import pallas as pl
from jax.experimental.pallas import tpu as pltpu

C = 128
H = 256
F = 256
ITERS = 2
ET = 4
NSEM = 57
NCHUNK = 8
CB = C // NCHUNK
COLS = CB * H
IB = 16


def _body(parent_ref, wp_hbm, bp_ref, wex_ref, bex_ref, wel_ref, bel_ref,
          wee_ref, bee_ref, wne_ref, bne_ref, wch_ref, bch_ref,
          wsem_ref, bsem_ref, wch2_ref, bch2_ref,
          out_ref, sem_ref, exists_ref, elog_ref,
          wp_buf, child_ref, sems):
    for k in range(NCHUNK):
        pltpu.make_async_copy(wp_hbm.at[:, pl.ds(k * COLS, COLS)],
                              wp_buf.at[k], sems.at[k]).start()
    for k in range(NCHUNK):
        pltpu.make_async_copy(wp_hbm.at[:, pl.ds(k * COLS, COLS)],
                              wp_buf.at[k], sems.at[k]).wait()
        pf = jnp.dot(parent_ref[...], wp_buf[k],
                     preferred_element_type=jnp.float32) \
            + bp_ref[pl.ds(k * COLS, COLS)][None, :]
        child_ref[pl.ds(k * CB, CB), :] = jnp.maximum(pf, 0.0).reshape(CB, H)

    if True:
        child = child_ref[...]

        exl = jnp.dot(child, wex_ref[...],
                      preferred_element_type=jnp.float32) + bex_ref[...]
        exists_ref[...] = exl[None]
        exf = (exl[:, 0] > 0.0).astype(jnp.float32)

        wel = wel_ref[...]
        ea = jnp.dot(child, wel[:H],
                     preferred_element_type=jnp.float32) + bel_ref[...]
        eb = jnp.dot(child, wel[H:], preferred_element_type=jnp.float32)
        cnt_rows = []
        for ib in range(C // IB):
            el = jnp.maximum(ea[ib * IB:(ib + 1) * IB][:, None, :]
                             + eb[None, :, :], 0.0)
            lb = jnp.dot(el.reshape(IB * C, H), wee_ref[...],
                         preferred_element_type=jnp.float32) + bee_ref[...]
            lb3 = lb.reshape(IB, C, ET)
            elog_ref[0, pl.ds(ib * IB, IB), :, :] = lb3
            pos = (lb3 > 0.0).astype(jnp.float32).sum(axis=2)
            cnt_rows.append(pos * exf[ib * IB:(ib + 1) * IB][:, None]
                            * exf[None, :])
        cnt = jnp.concatenate(cnt_rows, axis=0)
        has_edges = jnp.any(cnt > 0.0)

        cf = child
        feats = [child]
        for it in range(ITERS):
            a = jnp.dot(cf, wne_ref[it, :H],
                        preferred_element_type=jnp.float32) + bne_ref[it][None, :]
            b = jnp.dot(cf, wne_ref[it, H:], preferred_element_type=jnp.float32)
            rows = []
            for ib in range(C // IB):
                m = jnp.maximum(a[ib * IB:(ib + 1) * IB][:, None, :]
                                + b[None, :, :], 0.0)
                w = cnt[ib * IB:(ib + 1) * IB][:, :, None]
                rows.append(jnp.sum(m * w, axis=1))
            agg = jnp.concatenate(rows, axis=0)
            cf = jnp.where(has_edges, agg, cf)
            feats.append(cf)

        cf3 = jnp.concatenate(feats, axis=1)
        h = jnp.maximum(jnp.dot(cf3, wch_ref[...],
                                preferred_element_type=jnp.float32)
                        + bch_ref[...], 0.0)
        sem_ref[...] = (jnp.dot(h, wsem_ref[...],
                                preferred_element_type=jnp.float32)
                        + bsem_ref[...])[None]
        out_ref[...] = jnp.maximum(jnp.dot(h, wch2_ref[...],
                                           preferred_element_type=jnp.float32)
                                   + bch2_ref[...], 0.0)[None]


def kernel(parent_feature, W_parent, b_parent, W_exists, b_exists, W_el, b_el,
           W_ee, b_ee, W_ne, b_ne, W_child, b_child, W_sem, b_sem,
           W_child2, b_child2):
    f32 = jnp.float32
    wee2 = W_ee[:, :, 0].T
    bee2 = b_ee[:, 0][None, :]
    full = lambda s: pl.BlockSpec(s, lambda *_: (0,) * len(s))
    out, sem, exists_logits, elog = pl.pallas_call(
        _body,
        in_specs=[
            full((1, F)),
            pl.BlockSpec(memory_space=pl.ANY),
            full((C * H,)),
            full((H, 1)), full((1, 1)),
            full((2 * H, H)), full((1, H)),
            full((H, ET)), full((1, ET)),
            full((ITERS, 2 * H, H)), full((ITERS, H)),
            full((H * (ITERS + 1), H)), full((1, H)),
            full((H, NSEM)), full((1, NSEM)),
            full((H, F)), full((1, F)),
        ],
        out_specs=[
            full((1, C, F)), full((1, C, NSEM)),
            full((1, C, 1)), full((1, C, C, ET)),
        ],
        out_shape=[
            jax.ShapeDtypeStruct((1, C, F), f32),
            jax.ShapeDtypeStruct((1, C, NSEM), f32),
            jax.ShapeDtypeStruct((1, C, 1), f32),
            jax.ShapeDtypeStruct((1, C, C, ET), f32),
        ],
        scratch_shapes=[pltpu.VMEM((NCHUNK, F, COLS), f32),
                        pltpu.VMEM((C, H), f32),
                        pltpu.SemaphoreType.DMA((NCHUNK,))],
    )(parent_feature, W_parent, b_parent,
      W_exists, b_exists[None, :],
      W_el, b_el[None, :],
      wee2, bee2,
      W_ne, b_ne,
      W_child, b_child[None, :],
      W_sem, b_sem[None, :],
      W_child2, b_child2[None, :])
    return out, sem, exists_logits, elog
